# Initial kernel scaffold; baseline (speedup 1.0000x reference)
import jax
import jax.numpy as jnp
from jax import lax
from jax.experimental import pallas as pl
from jax.experimental.pallas import tpu as pltpu
from jax.experimental.pallas import tpu_sc as plsc

N = 100000
F = 128
B = 1024

NC = 2
NS = 16
L = 16
FW = F // NC

RB = 128
NFULL = N // RB
TAIL = N - NFULL * RB
TAIL_BASE = N - RB
NCHUNK = NFULL + 1
CPS = -(-NCHUNK // NS)
DUMP = B

ROWS_PER_TILE = B // NS


def _body(feat_hbm, mem_hbm, out_hbm, idx_v, rows_v, zbuf, acc_sh, sem):
    cid = lax.axis_index("c")
    sid = lax.axis_index("s")
    col0 = cid * FW

    def zero_row(r, _):
        def zero_col(k, _):
            zbuf[r, pl.ds(k * L, L)] = jnp.zeros((L,), jnp.float32)
            return 0
        return lax.fori_loop(0, FW // L, zero_col, 0)

    lax.fori_loop(0, ROWS_PER_TILE, zero_row, 0)
    pltpu.sync_copy(zbuf, acc_sh.at[pl.ds(sid * ROWS_PER_TILE, ROWS_PER_TILE)])
    plsc.subcore_barrier()

    def chunk_step(j, _):
        ck = sid + j * NS

        @pl.when(ck < NCHUNK)
        def _():
            is_tail = ck == NCHUNK - 1
            row0 = jnp.where(is_tail, TAIL_BASE, ck * RB)
            pltpu.sync_copy(mem_hbm.at[pl.ds(row0, RB)], idx_v)
            pltpu.sync_copy(
                feat_hbm.at[pl.ds(row0, RB), pl.ds(col0, FW)], rows_v
            )

            @pl.when(is_tail)
            def _():
                for t in range((RB - TAIL) // L):
                    idx_v[pl.ds(t * L, L)] = jnp.full((L,), DUMP, jnp.int32)

            pltpu.sync_copy(rows_v, acc_sh.at[idx_v], add=True)

        return 0

    lax.fori_loop(0, CPS, chunk_step, 0)
    plsc.subcore_barrier()

    r0 = sid * ROWS_PER_TILE
    pltpu.sync_copy(
        acc_sh.at[pl.ds(r0, ROWS_PER_TILE)],
        out_hbm.at[pl.ds(r0, ROWS_PER_TILE), pl.ds(col0, FW)],
    )


_segsum = pl.kernel(
    _body,
    out_type=jax.ShapeDtypeStruct((B, F), jnp.float32),
    mesh=plsc.VectorSubcoreMesh(core_axis_name="c", subcore_axis_name="s"),
    scratch_types=[
        pltpu.VMEM((RB,), jnp.int32),
        pltpu.VMEM((RB, FW), jnp.float32),
        pltpu.VMEM((ROWS_PER_TILE, FW), jnp.float32),
        pltpu.VMEM_SHARED((B + 8, FW), jnp.float32),
        pltpu.SemaphoreType.DMA,
    ],
)


@jax.jit
def kernel(atom_features, deg_slice, membership):
    del deg_slice
    return _segsum(atom_features, membership.astype(jnp.int32))


# speedup vs baseline: 3.3776x; 3.3776x over previous
import jax
import jax.numpy as jnp
from jax import lax
from jax.experimental import pallas as pl
from jax.experimental.pallas import tpu as pltpu
from jax.experimental.pallas import tpu_sc as plsc

N = 100000
F = 128
B = 1024

NC = 2
NS = 16
L = 16
FW = F // NC

RB = 128
NFULL = N // RB
TAIL = N - NFULL * RB
TAIL_BASE = N - RB
NCHUNK = NFULL + 1
CPS = -(-NCHUNK // NS)
DUMP = B

ROWS_PER_TILE = B // NS


def _body(feat_hbm, mem_hbm, out_hbm, idx_v, rows_v, zbuf, acc_sh, sem):
    cid = lax.axis_index("c")
    sid = lax.axis_index("s")
    col0 = cid * FW

    def zero_row(r, _):
        def zero_col(k, _):
            zbuf[r, pl.ds(k * L, L)] = jnp.zeros((L,), jnp.float32)
            return 0
        return lax.fori_loop(0, FW // L, zero_col, 0)

    lax.fori_loop(0, ROWS_PER_TILE, zero_row, 0)
    pltpu.sync_copy(zbuf, acc_sh.at[pl.ds(sid * ROWS_PER_TILE, ROWS_PER_TILE)])
    plsc.subcore_barrier()

    def chunk_step(j, _):
        ck = sid + j * NS

        @pl.when(ck < NCHUNK)
        def _():
            is_tail = ck == NCHUNK - 1
            row0 = jnp.where(is_tail, TAIL_BASE, ck * RB)
            pltpu.sync_copy(mem_hbm.at[pl.ds(row0, RB)], idx_v)
            pltpu.sync_copy(
                feat_hbm.at[pl.ds(row0, RB), pl.ds(col0, FW)], rows_v
            )

            @pl.when(is_tail)
            def _():
                for t in range((RB - TAIL) // L):
                    idx_v[pl.ds(t * L, L)] = jnp.full((L,), DUMP, jnp.int32)

            pltpu.sync_copy(rows_v, acc_sh.at[idx_v], add=True)

        return 0

    lax.fori_loop(0, CPS, chunk_step, 0)
    plsc.subcore_barrier()

    r0 = sid * ROWS_PER_TILE
    pltpu.sync_copy(
        acc_sh.at[pl.ds(r0, ROWS_PER_TILE)],
        out_hbm.at[pl.ds(r0, ROWS_PER_TILE), pl.ds(col0, FW)],
    )


_segsum = pl.kernel(
    _body,
    out_type=jax.ShapeDtypeStruct((B, F), jnp.float32),
    mesh=plsc.VectorSubcoreMesh(core_axis_name="c", subcore_axis_name="s"),
    scratch_types=[
        pltpu.VMEM((RB,), jnp.int32),
        pltpu.VMEM((RB, FW), jnp.float32),
        pltpu.VMEM((ROWS_PER_TILE, FW), jnp.float32),
        pltpu.VMEM_SHARED((B + 8, FW), jnp.float32),
        pltpu.SemaphoreType.DMA,
    ],
    compiler_params=pltpu.CompilerParams(use_tc_tiling_on_sc=False),
)


@jax.jit
def kernel(atom_features, deg_slice, membership):
    del deg_slice
    return _segsum(atom_features, membership.astype(jnp.int32))


# speedup vs baseline: 5.7731x; 1.7092x over previous
import jax
import jax.numpy as jnp
from jax import lax
from jax.experimental import pallas as pl
from jax.experimental.pallas import tpu as pltpu
from jax.experimental.pallas import tpu_sc as plsc

N = 100000
F = 128
B = 1024

NC = 2
NS = 16
L = 16
FW = F // NC

RB = 128
MAIN = N // RB
TAIL = N - MAIN * RB
TAIL_BASE = N - RB
DUMP = B

CPT = -(-MAIN // NS)
G = 4
SLAB = G * RB
SLOTS = 48 // G

ROWS_PER_TILE = B // NS


def _body(feat_hbm, mem_hbm, mem2d_hbm, out_hbm,
          idxs, idx1, rows0, rows1, zbuf, acc_sh,
          sem_ld0, sem_ld1, sem_add0, sem_add1):
    cid = lax.axis_index("c")
    sid = lax.axis_index("s")
    col0 = cid * FW
    start = sid * CPT - jnp.maximum(sid - 13, 0)

    rows = (rows0, rows1)
    sem_ld = (sem_ld0, sem_ld1)
    sem_add = (sem_add0, sem_add1)

    def zero_row(r, _):
        def zero_col(k, _):
            zbuf[r, pl.ds(k * L, L)] = jnp.zeros((L,), jnp.float32)
            return 0
        return lax.fori_loop(0, FW // L, zero_col, 0)

    lax.fori_loop(0, ROWS_PER_TILE, zero_row, 0)
    pltpu.sync_copy(zbuf, acc_sh.at[pl.ds(sid * ROWS_PER_TILE, ROWS_PER_TILE)])
    pltpu.sync_copy(mem2d_hbm.at[pl.ds(start, CPT)], idxs)
    plsc.subcore_barrier()

    def load_slab(b, s):
        row0 = (start + G * s) * RB
        pltpu.async_copy(
            feat_hbm.at[pl.ds(row0, SLAB), pl.ds(col0, FW)], rows[b], sem_ld[b]
        )

    def wait_load(b):
        pltpu.make_async_copy(
            feat_hbm.at[pl.ds(0, SLAB), pl.ds(col0, FW)], rows[b], sem_ld[b]
        ).wait()

    def issue_adds(b, s):
        for g in range(G):
            pltpu.async_copy(
                rows[b].at[pl.ds(g * RB, RB)],
                acc_sh.at[idxs.at[G * s + g]],
                sem_add[b],
                add=True,
            )

    def wait_adds(b):
        for g in range(G):
            pltpu.make_async_copy(
                rows[b].at[pl.ds(g * RB, RB)],
                acc_sh.at[idxs.at[0]],
                sem_add[b],
            ).wait()

    load_slab(0, 0)
    wait_load(0)
    issue_adds(0, 0)
    load_slab(1, 1)

    def slot_pair(j2, _):
        s1 = 1 + 2 * j2
        wait_load(1)
        issue_adds(1, s1)
        wait_adds(0)
        load_slab(0, s1 + 1)
        wait_load(0)
        issue_adds(0, s1 + 1)
        wait_adds(1)
        load_slab(1, s1 + 2)
        return 0

    lax.fori_loop(0, (SLOTS - 2) // 2, slot_pair, 0)
    wait_load(1)
    issue_adds(1, SLOTS - 1)
    wait_adds(0)
    wait_adds(1)

    @pl.when(sid <= 12)
    def _():
        row0 = (start + 48) * RB
        pltpu.sync_copy(
            feat_hbm.at[pl.ds(row0, RB), pl.ds(col0, FW)],
            rows0.at[pl.ds(0, RB)],
        )
        pltpu.sync_copy(
            rows0.at[pl.ds(0, RB)], acc_sh.at[idxs.at[48]], add=True
        )

    @pl.when(sid == 15)
    def _():
        pltpu.sync_copy(mem_hbm.at[pl.ds(TAIL_BASE, RB)], idx1)
        for t in range((RB - TAIL) // L):
            idx1[pl.ds(t * L, L)] = jnp.full((L,), DUMP, jnp.int32)
        pltpu.sync_copy(
            feat_hbm.at[pl.ds(TAIL_BASE, RB), pl.ds(col0, FW)],
            rows0.at[pl.ds(0, RB)],
        )
        pltpu.sync_copy(rows0.at[pl.ds(0, RB)], acc_sh.at[idx1], add=True)

    plsc.subcore_barrier()

    r0 = sid * ROWS_PER_TILE
    pltpu.sync_copy(
        acc_sh.at[pl.ds(r0, ROWS_PER_TILE)],
        out_hbm.at[pl.ds(r0, ROWS_PER_TILE), pl.ds(col0, FW)],
    )


_segsum = pl.kernel(
    _body,
    out_type=jax.ShapeDtypeStruct((B, F), jnp.float32),
    mesh=plsc.VectorSubcoreMesh(core_axis_name="c", subcore_axis_name="s"),
    scratch_types=[
        pltpu.VMEM((CPT, RB), jnp.int32),
        pltpu.VMEM((RB,), jnp.int32),
        pltpu.VMEM((SLAB, FW), jnp.float32),
        pltpu.VMEM((SLAB, FW), jnp.float32),
        pltpu.VMEM((ROWS_PER_TILE, FW), jnp.float32),
        pltpu.VMEM_SHARED((B + 8, FW), jnp.float32),
        pltpu.SemaphoreType.DMA,
        pltpu.SemaphoreType.DMA,
        pltpu.SemaphoreType.DMA,
        pltpu.SemaphoreType.DMA,
    ],
    compiler_params=pltpu.CompilerParams(use_tc_tiling_on_sc=False),
)


@jax.jit
def kernel(atom_features, deg_slice, membership):
    del deg_slice
    m32 = membership.astype(jnp.int32)
    mem2d = jnp.pad(m32[: MAIN * RB].reshape(MAIN, RB), ((0, 3), (0, 0)))
    return _segsum(atom_features, m32, mem2d)


# speedup vs baseline: 6.4392x; 1.1154x over previous
import jax
import jax.numpy as jnp
from jax import lax
from jax.experimental import pallas as pl
from jax.experimental.pallas import tpu as pltpu
from jax.experimental.pallas import tpu_sc as plsc

N = 100000
F = 128
B = 1024

NC = 2
NS = 16
NW = NC * NS
L = 16

RB = 128
MAIN = N // RB
TAIL = N - MAIN * RB
TAIL_BASE = N - RB
DUMP = B

CPT = 25
G = 3
SLAB = G * RB
SLOTS = 24 // G

ROWS_PER_TILE = B // NS


def _body(feat_hbm, mem_hbm, mem2d_hbm, out_hbm,
          idxs, idx1, rows0, rows1, zbuf, acc_sh,
          sem_ld0, sem_ld1, sem_add0, sem_add1):
    cid = lax.axis_index("c")
    sid = lax.axis_index("s")
    wid = cid * NS + sid
    start = wid * CPT - jnp.maximum(wid - 13, 0)

    rows = (rows0, rows1)
    sem_ld = (sem_ld0, sem_ld1)
    sem_add = (sem_add0, sem_add1)

    def zero_row(r, _):
        def zero_col(k, _):
            zbuf[r, pl.ds(k * L, L)] = jnp.zeros((L,), jnp.float32)
            return 0
        return lax.fori_loop(0, F // L, zero_col, 0)

    lax.fori_loop(0, ROWS_PER_TILE, zero_row, 0)
    pltpu.sync_copy(zbuf, acc_sh.at[pl.ds(sid * ROWS_PER_TILE, ROWS_PER_TILE)])
    pltpu.sync_copy(mem2d_hbm.at[pl.ds(start, CPT)], idxs)
    plsc.subcore_barrier()

    def load_slab(b, s):
        row0 = (start + G * s) * RB
        pltpu.async_copy(feat_hbm.at[pl.ds(row0, SLAB), :], rows[b], sem_ld[b])

    def wait_load(b):
        pltpu.make_async_copy(
            feat_hbm.at[pl.ds(0, SLAB), :], rows[b], sem_ld[b]
        ).wait()

    def issue_adds(b, s):
        for g in range(G):
            pltpu.async_copy(
                rows[b].at[pl.ds(g * RB, RB)],
                acc_sh.at[idxs.at[G * s + g]],
                sem_add[b],
                add=True,
            )

    def wait_adds(b):
        for g in range(G):
            pltpu.make_async_copy(
                rows[b].at[pl.ds(g * RB, RB)],
                acc_sh.at[idxs.at[0]],
                sem_add[b],
            ).wait()

    load_slab(0, 0)
    wait_load(0)
    issue_adds(0, 0)
    load_slab(1, 1)

    def slot_pair(j2, _):
        s1 = 1 + 2 * j2
        wait_load(1)
        issue_adds(1, s1)
        wait_adds(0)
        load_slab(0, s1 + 1)
        wait_load(0)
        issue_adds(0, s1 + 1)
        wait_adds(1)
        load_slab(1, s1 + 2)
        return 0

    lax.fori_loop(0, (SLOTS - 2) // 2, slot_pair, 0)
    wait_load(1)
    issue_adds(1, SLOTS - 1)
    wait_adds(0)
    wait_adds(1)

    @pl.when(wid <= 12)
    def _():
        row0 = (start + 24) * RB
        pltpu.sync_copy(
            feat_hbm.at[pl.ds(row0, RB), :], rows0.at[pl.ds(0, RB)]
        )
        pltpu.sync_copy(
            rows0.at[pl.ds(0, RB)], acc_sh.at[idxs.at[24]], add=True
        )

    @pl.when(wid == NW - 1)
    def _():
        pltpu.sync_copy(mem_hbm.at[pl.ds(TAIL_BASE, RB)], idx1)
        for t in range((RB - TAIL) // L):
            idx1[pl.ds(t * L, L)] = jnp.full((L,), DUMP, jnp.int32)
        pltpu.sync_copy(
            feat_hbm.at[pl.ds(TAIL_BASE, RB), :], rows0.at[pl.ds(0, RB)]
        )
        pltpu.sync_copy(rows0.at[pl.ds(0, RB)], acc_sh.at[idx1], add=True)

    plsc.subcore_barrier()

    r0 = sid * ROWS_PER_TILE
    pltpu.sync_copy(
        acc_sh.at[pl.ds(r0, ROWS_PER_TILE)],
        out_hbm.at[cid, pl.ds(r0, ROWS_PER_TILE), :],
    )


_segsum = pl.kernel(
    _body,
    out_type=jax.ShapeDtypeStruct((NC, B, F), jnp.float32),
    mesh=plsc.VectorSubcoreMesh(core_axis_name="c", subcore_axis_name="s"),
    scratch_types=[
        pltpu.VMEM((CPT, RB), jnp.int32),
        pltpu.VMEM((RB,), jnp.int32),
        pltpu.VMEM((SLAB, F), jnp.float32),
        pltpu.VMEM((SLAB, F), jnp.float32),
        pltpu.VMEM((ROWS_PER_TILE, F), jnp.float32),
        pltpu.VMEM_SHARED((B + 8, F), jnp.float32),
        pltpu.SemaphoreType.DMA,
        pltpu.SemaphoreType.DMA,
        pltpu.SemaphoreType.DMA,
        pltpu.SemaphoreType.DMA,
    ],
    compiler_params=pltpu.CompilerParams(use_tc_tiling_on_sc=False),
)


@jax.jit
def kernel(atom_features, deg_slice, membership):
    del deg_slice
    m32 = membership.astype(jnp.int32)
    mem2d = jnp.pad(m32[: MAIN * RB].reshape(MAIN, RB), ((0, 3), (0, 0)))
    partials = _segsum(atom_features, m32, mem2d)
    return partials[0] + partials[1]
